# baseline (speedup 1.0000x reference)
import functools

import jax
import jax.numpy as jnp
from jax import lax
from jax.experimental import pallas as pl
from jax.experimental.pallas import tpu as pltpu
from jax.experimental.pallas import tpu_sc as plsc

N = 10000
E = 320000
IN = 128
HID = 128
LAT = 64

NC = 2
NS = 16
NW = NC * NS
CHUNK = 128
CPT = 80
NBUF = 2
E_PAD = NW * CPT * CHUNK
N_PAD = 10112
RPT = N_PAD // NS
EPS = 1e-5

_mesh = plsc.VectorSubcoreMesh(core_axis_name="c", subcore_axis_name="s")


@functools.partial(
    pl.kernel,
    mesh=_mesh,
    out_type=jax.ShapeDtypeStruct((NC, N_PAD, HID), jnp.float32),
    scratch_types=[
        pltpu.VMEM((CHUNK,), jnp.int32),
        pltpu.VMEM((CHUNK, HID), jnp.float32),
        pltpu.VMEM_SHARED((N_PAD, HID), jnp.float32),
    ],
)
def _deg_kernel(dst_hbm, ones_hbm, zero_hbm, out_hbm, didx, ones_v, acc):
    cid = lax.axis_index("c")
    sid = lax.axis_index("s")
    wid = cid * NS + sid
    r0 = sid * RPT
    pltpu.sync_copy(ones_hbm, ones_v)
    pltpu.sync_copy(zero_hbm.at[pl.ds(r0, RPT)], acc.at[pl.ds(r0, RPT)])
    plsc.subcore_barrier()

    def body(j, carry):
        base = (wid * CPT + j) * CHUNK
        pltpu.sync_copy(dst_hbm.at[pl.ds(base, CHUNK)], didx)
        pltpu.sync_copy(ones_v, acc.at[didx], add=True)
        return carry

    lax.fori_loop(0, CPT, body, 0)
    plsc.subcore_barrier()
    pltpu.sync_copy(acc.at[pl.ds(r0, RPT)], out_hbm.at[cid].at[pl.ds(r0, RPT)])


def _make_scatter(F):
    @functools.partial(
        pl.kernel,
        mesh=_mesh,
        out_type=jax.ShapeDtypeStruct((NC, N_PAD, F), jnp.float32),
        scratch_types=[
            pltpu.VMEM((CHUNK,), jnp.int32),
            pltpu.VMEM((CHUNK,), jnp.int32),
            pltpu.VMEM((CHUNK, F), jnp.float32),
            pltpu.VMEM_SHARED((N_PAD, F), jnp.float32),
        ],
    )
    def _scatter_kernel(h_hbm, src_hbm, dst_hbm, zero_hbm, out_hbm,
                        sidx, didx, rows, acc):
        cid = lax.axis_index("c")
        sid = lax.axis_index("s")
        wid = cid * NS + sid
        r0 = sid * RPT
        pltpu.sync_copy(zero_hbm.at[pl.ds(r0, RPT)], acc.at[pl.ds(r0, RPT)])
        plsc.subcore_barrier()

        def body(j, carry):
            base = (wid * CPT + j) * CHUNK
            pltpu.sync_copy(src_hbm.at[pl.ds(base, CHUNK)], sidx)
            pltpu.sync_copy(dst_hbm.at[pl.ds(base, CHUNK)], didx)
            pltpu.sync_copy(h_hbm.at[sidx], rows)
            pltpu.sync_copy(rows, acc.at[didx], add=True)
            return carry

        lax.fori_loop(0, CPT, body, 0)
        plsc.subcore_barrier()
        pltpu.sync_copy(acc.at[pl.ds(r0, RPT)], out_hbm.at[cid].at[pl.ds(r0, RPT)])

    return _scatter_kernel


_scatter_hid = _make_scatter(HID)


def _dinv_from(d0_ref, d1_ref):
    deg = d0_ref[:, 0:1] + d1_ref[:, 0:1] + 1.0
    return 1.0 / jnp.sqrt(deg)


def _first_body(d0_ref, d1_ref, x_ref, w_ref, out_ref):
    dinv = _dinv_from(d0_ref, d1_ref)
    out_ref[...] = (x_ref[...] @ w_ref[...]) * dinv


def _mid_body(d0_ref, d1_ref, sp_ref, hp_ref, b_ref, g_ref, be_ref, w_ref,
              out_ref):
    dinv = _dinv_from(d0_ref, d1_ref)
    y = dinv * (sp_ref[0] + sp_ref[1] + hp_ref[...]) + b_ref[...]
    yr = y[:N]
    m = jnp.mean(yr, axis=0, keepdims=True)
    v = jnp.mean((yr - m) ** 2, axis=0, keepdims=True)
    a = (y - m) / jnp.sqrt(v + EPS) * g_ref[...] + be_ref[...]
    a = jnp.maximum(a, 0.0)
    rows = lax.broadcasted_iota(jnp.int32, a.shape, 0)
    a = jnp.where(rows < N, a, 0.0)
    out_ref[...] = (a @ w_ref[...]) * dinv


def _bn_relu(y, g, b):
    m = jnp.mean(y, axis=0, keepdims=True)
    v = jnp.mean((y - m) ** 2, axis=0, keepdims=True)
    return jnp.maximum((y - m) / jnp.sqrt(v + EPS) * g + b, 0.0)


def _dec_body(dg0_ref, dg1p_ref, sp_ref, hp_ref, b3_ref,
              d1_ref, db1_ref, dg1_ref, dbe1_ref,
              d2_ref, db2_ref, dg2_ref, dbe2_ref,
              d3_ref, db3_ref, xhat_ref, z_ref):
    dinv = _dinv_from(dg0_ref, dg1p_ref)
    zfull = dinv * (sp_ref[0] + sp_ref[1] + hp_ref[...])
    z = zfull[:N, :LAT] + b3_ref[...]
    z_ref[...] = z
    d = _bn_relu(z @ d1_ref[...] + db1_ref[...], dg1_ref[...], dbe1_ref[...])
    d = _bn_relu(d @ d2_ref[...] + db2_ref[...], dg2_ref[...], dbe2_ref[...])
    xhat_ref[...] = d @ d3_ref[...] + db3_ref[...]


def _tc_call(body, out_shape, *args):
    return pl.pallas_call(body, out_shape=out_shape)(*args)


def kernel(x, edge_index, W1, b1, g1, be1, W2, b2, g2, be2, W3, b3,
           D1, db1, dg1, dbe1, D2, db2, dg2, dbe2, D3, db3):
    pad_e = E_PAD - E
    src_p = jnp.concatenate([edge_index[0], jnp.full((pad_e,), N, jnp.int32)])
    dst_p = jnp.concatenate([edge_index[1], jnp.full((pad_e,), N, jnp.int32)])
    x_p = jnp.concatenate([x, jnp.zeros((N_PAD - N, IN), x.dtype)], axis=0)
    z128 = jnp.zeros((N_PAD, HID), jnp.float32)
    ones128 = jnp.ones((CHUNK, HID), jnp.float32)
    r1 = lambda a: a.reshape(1, -1)

    degp = _deg_kernel(dst_p, ones128, z128)
    d0, d1 = degp[0], degp[1]

    h1p = _tc_call(_first_body, jax.ShapeDtypeStruct((N_PAD, HID), jnp.float32),
                   d0, d1, x_p, W1)
    s1 = _scatter_hid(h1p, src_p, dst_p, z128)
    h2p = _tc_call(_mid_body, jax.ShapeDtypeStruct((N_PAD, HID), jnp.float32),
                   d0, d1, s1, h1p, r1(b1), r1(g1), r1(be1), W2)
    s2 = _scatter_hid(h2p, src_p, dst_p, z128)
    W3p = jnp.pad(W3, ((0, 0), (0, HID - LAT)))
    h3p = _tc_call(_mid_body, jax.ShapeDtypeStruct((N_PAD, HID), jnp.float32),
                   d0, d1, s2, h2p, r1(b2), r1(g2), r1(be2), W3p)
    s3 = _scatter_hid(h3p, src_p, dst_p, z128)
    xhat, z = _tc_call(
        _dec_body,
        (jax.ShapeDtypeStruct((N, IN), jnp.float32),
         jax.ShapeDtypeStruct((N, LAT), jnp.float32)),
        d0, d1, s3, h3p, r1(b3),
        D1, r1(db1), r1(dg1), r1(dbe1),
        D2, r1(db2), r1(dg2), r1(dbe2),
        D3, r1(db3))
    return (xhat, z)


# speedup vs baseline: 1.0002x; 1.0002x over previous
import functools

import jax
import jax.numpy as jnp
from jax import lax
from jax.experimental import pallas as pl
from jax.experimental.pallas import tpu as pltpu
from jax.experimental.pallas import tpu_sc as plsc

N = 10000
E = 320000
IN = 128
HID = 128
LAT = 64

NC = 2
NS = 16
NW = NC * NS
CHUNK = 128
CPT = 80
NBUF = 2
E_PAD = NW * CPT * CHUNK
N_PAD = 10112
RPT = N_PAD // NS
EPS = 1e-5

_mesh = plsc.VectorSubcoreMesh(core_axis_name="c", subcore_axis_name="s")


@functools.partial(
    pl.kernel,
    mesh=_mesh,
    out_type=jax.ShapeDtypeStruct((NC, N_PAD, HID), jnp.float32),
    scratch_types=[
        pltpu.VMEM((CHUNK,), jnp.int32),
        pltpu.VMEM((CHUNK, HID), jnp.float32),
        pltpu.VMEM_SHARED((N_PAD, HID), jnp.float32),
    ],
)
def _deg_kernel(dst_hbm, ones_hbm, zero_hbm, out_hbm, didx, ones_v, acc):
    cid = lax.axis_index("c")
    sid = lax.axis_index("s")
    wid = cid * NS + sid
    r0 = sid * RPT
    pltpu.sync_copy(ones_hbm, ones_v)
    pltpu.sync_copy(zero_hbm.at[pl.ds(r0, RPT)], acc.at[pl.ds(r0, RPT)])
    plsc.subcore_barrier()

    def body(j, carry):
        base = (wid * CPT + j) * CHUNK
        pltpu.sync_copy(dst_hbm.at[pl.ds(base, CHUNK)], didx)
        pltpu.sync_copy(ones_v, acc.at[didx], add=True)
        return carry

    lax.fori_loop(0, CPT, body, 0)
    plsc.subcore_barrier()
    pltpu.sync_copy(acc.at[pl.ds(r0, RPT)], out_hbm.at[cid].at[pl.ds(r0, RPT)])


def _make_scatter(F):
    @functools.partial(
        pl.kernel,
        mesh=_mesh,
        out_type=jax.ShapeDtypeStruct((NC, N_PAD, F), jnp.float32),
        scratch_types=[
            pltpu.VMEM((CHUNK,), jnp.int32),
            pltpu.VMEM((CHUNK,), jnp.int32),
            pltpu.VMEM((CHUNK, F), jnp.float32),
            pltpu.VMEM_SHARED((N_PAD, F), jnp.float32),
            pltpu.SemaphoreType.DMA,
        ],
    )
    def _scatter_kernel(h_hbm, src_hbm, dst_hbm, zero_hbm, out_hbm,
                        sidx, didx, rows, acc, sem):
        cid = lax.axis_index("c")
        sid = lax.axis_index("s")
        wid = cid * NS + sid
        r0 = sid * RPT
        pltpu.sync_copy(zero_hbm.at[pl.ds(r0, RPT)], acc.at[pl.ds(r0, RPT)])
        plsc.subcore_barrier()

        def body(j, carry):
            base = (wid * CPT + j) * CHUNK
            pltpu.sync_copy(src_hbm.at[pl.ds(base, CHUNK)], sidx)
            pltpu.sync_copy(dst_hbm.at[pl.ds(base, CHUNK)], didx)
            pltpu.async_copy(h_hbm.at[sidx], rows, sem).wait()
            pltpu.sync_copy(rows, acc.at[didx], add=True)
            return carry

        lax.fori_loop(0, CPT, body, 0)
        plsc.subcore_barrier()
        pltpu.sync_copy(acc.at[pl.ds(r0, RPT)], out_hbm.at[cid].at[pl.ds(r0, RPT)])

    return _scatter_kernel


_scatter_hid = _make_scatter(HID)


def _dinv_from(d0_ref, d1_ref):
    deg = d0_ref[:, 0:1] + d1_ref[:, 0:1] + 1.0
    return 1.0 / jnp.sqrt(deg)


def _first_body(d0_ref, d1_ref, x_ref, w_ref, out_ref):
    dinv = _dinv_from(d0_ref, d1_ref)
    out_ref[...] = (x_ref[...] @ w_ref[...]) * dinv


def _mid_body(d0_ref, d1_ref, sp_ref, hp_ref, b_ref, g_ref, be_ref, w_ref,
              out_ref):
    dinv = _dinv_from(d0_ref, d1_ref)
    y = dinv * (sp_ref[0] + sp_ref[1] + hp_ref[...]) + b_ref[...]
    yr = y[:N]
    m = jnp.mean(yr, axis=0, keepdims=True)
    v = jnp.mean((yr - m) ** 2, axis=0, keepdims=True)
    a = (y - m) / jnp.sqrt(v + EPS) * g_ref[...] + be_ref[...]
    a = jnp.maximum(a, 0.0)
    rows = lax.broadcasted_iota(jnp.int32, a.shape, 0)
    a = jnp.where(rows < N, a, 0.0)
    out_ref[...] = (a @ w_ref[...]) * dinv


def _bn_relu(y, g, b):
    m = jnp.mean(y, axis=0, keepdims=True)
    v = jnp.mean((y - m) ** 2, axis=0, keepdims=True)
    return jnp.maximum((y - m) / jnp.sqrt(v + EPS) * g + b, 0.0)


def _dec_body(dg0_ref, dg1p_ref, sp_ref, hp_ref, b3_ref,
              d1_ref, db1_ref, dg1_ref, dbe1_ref,
              d2_ref, db2_ref, dg2_ref, dbe2_ref,
              d3_ref, db3_ref, xhat_ref, z_ref):
    dinv = _dinv_from(dg0_ref, dg1p_ref)
    zfull = dinv * (sp_ref[0] + sp_ref[1] + hp_ref[...])
    z = zfull[:N, :LAT] + b3_ref[...]
    z_ref[...] = z
    d = _bn_relu(z @ d1_ref[...] + db1_ref[...], dg1_ref[...], dbe1_ref[...])
    d = _bn_relu(d @ d2_ref[...] + db2_ref[...], dg2_ref[...], dbe2_ref[...])
    xhat_ref[...] = d @ d3_ref[...] + db3_ref[...]


def _tc_call(body, out_shape, *args):
    return pl.pallas_call(body, out_shape=out_shape)(*args)


def kernel(x, edge_index, W1, b1, g1, be1, W2, b2, g2, be2, W3, b3,
           D1, db1, dg1, dbe1, D2, db2, dg2, dbe2, D3, db3):
    pad_e = E_PAD - E
    src_p = jnp.concatenate([edge_index[0], jnp.full((pad_e,), N, jnp.int32)])
    dst_p = jnp.concatenate([edge_index[1], jnp.full((pad_e,), N, jnp.int32)])
    x_p = jnp.concatenate([x, jnp.zeros((N_PAD - N, IN), x.dtype)], axis=0)
    z128 = jnp.zeros((N_PAD, HID), jnp.float32)
    ones128 = jnp.ones((CHUNK, HID), jnp.float32)
    r1 = lambda a: a.reshape(1, -1)

    degp = _deg_kernel(dst_p, ones128, z128)
    d0, d1 = degp[0], degp[1]

    h1p = _tc_call(_first_body, jax.ShapeDtypeStruct((N_PAD, HID), jnp.float32),
                   d0, d1, x_p, W1)
    s1 = _scatter_hid(h1p, src_p, dst_p, z128)
    h2p = _tc_call(_mid_body, jax.ShapeDtypeStruct((N_PAD, HID), jnp.float32),
                   d0, d1, s1, h1p, r1(b1), r1(g1), r1(be1), W2)
    s2 = _scatter_hid(h2p, src_p, dst_p, z128)
    W3p = jnp.pad(W3, ((0, 0), (0, HID - LAT)))
    h3p = _tc_call(_mid_body, jax.ShapeDtypeStruct((N_PAD, HID), jnp.float32),
                   d0, d1, s2, h2p, r1(b2), r1(g2), r1(be2), W3p)
    s3 = _scatter_hid(h3p, src_p, dst_p, z128)
    xhat, z = _tc_call(
        _dec_body,
        (jax.ShapeDtypeStruct((N, IN), jnp.float32),
         jax.ShapeDtypeStruct((N, LAT), jnp.float32)),
        d0, d1, s3, h3p, r1(b3),
        D1, r1(db1), r1(dg1), r1(dbe1),
        D2, r1(db2), r1(dg2), r1(dbe2),
        D3, r1(db3))
    return (xhat, z)


# speedup vs baseline: 2.0392x; 2.0389x over previous
import functools

import jax
import jax.numpy as jnp
from jax import lax
from jax.experimental import pallas as pl
from jax.experimental.pallas import tpu as pltpu
from jax.experimental.pallas import tpu_sc as plsc

N = 10000
E = 320000
IN = 128
HID = 128
LAT = 64

NC = 2
NS = 16
NW = NC * NS
CHUNK = 128
CPT = 80
NBUF = 2
E_PAD = NW * CPT * CHUNK
N_PAD = 10112
RPT = N_PAD // NS
EPS = 1e-5

_mesh = plsc.VectorSubcoreMesh(core_axis_name="c", subcore_axis_name="s")


@functools.partial(
    pl.kernel,
    mesh=_mesh,
    out_type=jax.ShapeDtypeStruct((NC, N_PAD, HID), jnp.float32),
    scratch_types=[
        pltpu.VMEM((CHUNK,), jnp.int32),
        pltpu.VMEM((CHUNK, HID), jnp.float32),
        pltpu.VMEM_SHARED((N_PAD, HID), jnp.float32),
    ],
)
def _deg_kernel(dst_hbm, ones_hbm, zero_hbm, out_hbm, didx, ones_v, acc):
    cid = lax.axis_index("c")
    sid = lax.axis_index("s")
    wid = cid * NS + sid
    r0 = sid * RPT
    pltpu.sync_copy(ones_hbm, ones_v)
    pltpu.sync_copy(zero_hbm.at[pl.ds(r0, RPT)], acc.at[pl.ds(r0, RPT)])
    plsc.subcore_barrier()

    def body(j, carry):
        base = (wid * CPT + j) * CHUNK
        pltpu.sync_copy(dst_hbm.at[pl.ds(base, CHUNK)], didx)
        pltpu.sync_copy(ones_v, acc.at[didx], add=True)
        return carry

    lax.fori_loop(0, CPT, body, 0)
    plsc.subcore_barrier()
    pltpu.sync_copy(acc.at[pl.ds(r0, RPT)], out_hbm.at[cid].at[pl.ds(r0, RPT)])


def _make_scatter(F):
    @functools.partial(
        pl.kernel,
        mesh=_mesh,
        out_type=jax.ShapeDtypeStruct((NC, N_PAD, F), jnp.float32),
        scratch_types=[
            pltpu.VMEM((CHUNK,), jnp.int32),
            pltpu.VMEM((CHUNK,), jnp.int32),
            pltpu.VMEM((CHUNK, F), jnp.float32),
            pltpu.VMEM_SHARED((N_PAD, F), jnp.float32),
            pltpu.SemaphoreType.DMA,
        ],
    )
    def _scatter_kernel(h_hbm, src_hbm, dst_hbm, zero_hbm, out_hbm,
                        sidx, didx, rows, acc, sem):
        cid = lax.axis_index("c")
        sid = lax.axis_index("s")
        wid = cid * NS + sid
        r0 = sid * RPT
        pltpu.sync_copy(zero_hbm.at[pl.ds(r0, RPT)], acc.at[pl.ds(r0, RPT)])
        plsc.subcore_barrier()

        def body(j, carry):
            base = (wid * CPT + j) * CHUNK
            pltpu.sync_copy(src_hbm.at[pl.ds(base, CHUNK)], sidx)
            pltpu.sync_copy(dst_hbm.at[pl.ds(base, CHUNK)], didx)
            pltpu.async_copy(h_hbm.at[sidx], rows, sem).wait()
            pltpu.sync_copy(rows, acc.at[didx], add=True)
            return carry

        lax.fori_loop(0, CPT, body, 0)
        plsc.subcore_barrier()
        pltpu.sync_copy(acc.at[pl.ds(r0, RPT)], out_hbm.at[cid].at[pl.ds(r0, RPT)])

    return _scatter_kernel


_scatter_hid = _make_scatter(HID)


def _dinv_from(d0_ref, d1_ref):
    deg = d0_ref[:, 0:1] + d1_ref[:, 0:1] + 1.0
    return 1.0 / jnp.sqrt(deg)


def _first_body(d0_ref, d1_ref, x_ref, w_ref, out_ref):
    dinv = _dinv_from(d0_ref, d1_ref)
    out_ref[...] = (x_ref[...] @ w_ref[...]) * dinv


def _mid_body(d0_ref, d1_ref, sp_ref, hp_ref, b_ref, g_ref, be_ref, w_ref,
              out_ref):
    dinv = _dinv_from(d0_ref, d1_ref)
    y = dinv * (sp_ref[0] + sp_ref[1] + hp_ref[...]) + b_ref[...]
    yr = y[:N]
    m = jnp.mean(yr, axis=0, keepdims=True)
    v = jnp.mean((yr - m) ** 2, axis=0, keepdims=True)
    a = (y - m) / jnp.sqrt(v + EPS) * g_ref[...] + be_ref[...]
    a = jnp.maximum(a, 0.0)
    rows = lax.broadcasted_iota(jnp.int32, a.shape, 0)
    a = jnp.where(rows < N, a, 0.0)
    out_ref[...] = (a @ w_ref[...]) * dinv


def _bn_relu(y, g, b):
    m = jnp.mean(y, axis=0, keepdims=True)
    v = jnp.mean((y - m) ** 2, axis=0, keepdims=True)
    return jnp.maximum((y - m) / jnp.sqrt(v + EPS) * g + b, 0.0)


def _dec_body(dg0_ref, dg1p_ref, sp_ref, hp_ref, b3_ref,
              d1_ref, db1_ref, dg1_ref, dbe1_ref,
              d2_ref, db2_ref, dg2_ref, dbe2_ref,
              d3_ref, db3_ref, xhat_ref, z_ref):
    dinv = _dinv_from(dg0_ref, dg1p_ref)
    zfull = dinv * (sp_ref[0] + sp_ref[1] + hp_ref[...])
    z = zfull[:N, :LAT] + b3_ref[...]
    z_ref[...] = z
    d = _bn_relu(z @ d1_ref[...] + db1_ref[...], dg1_ref[...], dbe1_ref[...])
    d = _bn_relu(d @ d2_ref[...] + db2_ref[...], dg2_ref[...], dbe2_ref[...])
    xhat_ref[...] = d @ d3_ref[...] + db3_ref[...]


def _tc_call(body, out_shape, *args):
    return pl.pallas_call(body, out_shape=out_shape)(*args)


def kernel(x, edge_index, W1, b1, g1, be1, W2, b2, g2, be2, W3, b3,
           D1, db1, dg1, dbe1, D2, db2, dg2, dbe2, D3, db3):
    pad_e = E_PAD - E
    pad_idx = N + (jnp.arange(pad_e, dtype=jnp.int32) % (N_PAD - N))
    src_p = jnp.concatenate([edge_index[0], pad_idx])
    dst_p = jnp.concatenate([edge_index[1], pad_idx])
    x_p = jnp.concatenate([x, jnp.zeros((N_PAD - N, IN), x.dtype)], axis=0)
    z128 = jnp.zeros((N_PAD, HID), jnp.float32)
    ones128 = jnp.ones((CHUNK, HID), jnp.float32)
    r1 = lambda a: a.reshape(1, -1)

    degp = _deg_kernel(dst_p, ones128, z128)
    d0, d1 = degp[0], degp[1]

    h1p = _tc_call(_first_body, jax.ShapeDtypeStruct((N_PAD, HID), jnp.float32),
                   d0, d1, x_p, W1)
    s1 = _scatter_hid(h1p, src_p, dst_p, z128)
    h2p = _tc_call(_mid_body, jax.ShapeDtypeStruct((N_PAD, HID), jnp.float32),
                   d0, d1, s1, h1p, r1(b1), r1(g1), r1(be1), W2)
    s2 = _scatter_hid(h2p, src_p, dst_p, z128)
    W3p = jnp.pad(W3, ((0, 0), (0, HID - LAT)))
    h3p = _tc_call(_mid_body, jax.ShapeDtypeStruct((N_PAD, HID), jnp.float32),
                   d0, d1, s2, h2p, r1(b2), r1(g2), r1(be2), W3p)
    s3 = _scatter_hid(h3p, src_p, dst_p, z128)
    xhat, z = _tc_call(
        _dec_body,
        (jax.ShapeDtypeStruct((N, IN), jnp.float32),
         jax.ShapeDtypeStruct((N, LAT), jnp.float32)),
        d0, d1, s3, h3p, r1(b3),
        D1, r1(db1), r1(dg1), r1(dbe1),
        D2, r1(db2), r1(dg2), r1(dbe2),
        D3, r1(db3))
    return (xhat, z)


# speedup vs baseline: 2.0410x; 1.0009x over previous
import functools

import jax
import jax.numpy as jnp
from jax import lax
from jax.experimental import pallas as pl
from jax.experimental.pallas import tpu as pltpu
from jax.experimental.pallas import tpu_sc as plsc

N = 10000
E = 320000
IN = 128
HID = 128
LAT = 64

NC = 2
NS = 16
NW = NC * NS
CHUNK = 128
CPT = 80
E_PAD = NW * CPT * CHUNK
N_PAD = 10112
RPT = N_PAD // NS
EPS = 1e-5

_mesh = plsc.VectorSubcoreMesh(core_axis_name="c", subcore_axis_name="s")


@functools.partial(
    pl.kernel,
    mesh=_mesh,
    out_type=jax.ShapeDtypeStruct((NC, N_PAD, HID), jnp.float32),
    scratch_types=[
        pltpu.VMEM((CHUNK,), jnp.int32),
        pltpu.VMEM((CHUNK, HID), jnp.float32),
        pltpu.VMEM_SHARED((N_PAD, HID), jnp.float32),
    ],
)
def _deg_kernel(dst_hbm, ones_hbm, zero_hbm, out_hbm, didx, ones_v, acc):
    cid = lax.axis_index("c")
    sid = lax.axis_index("s")
    wid = cid * NS + sid
    r0 = sid * RPT
    pltpu.sync_copy(ones_hbm, ones_v)
    pltpu.sync_copy(zero_hbm.at[pl.ds(r0, RPT)], acc.at[pl.ds(r0, RPT)])
    plsc.subcore_barrier()

    def body(j, carry):
        base = (wid * CPT + j) * CHUNK
        pltpu.sync_copy(dst_hbm.at[pl.ds(base, CHUNK)], didx)
        pltpu.sync_copy(ones_v, acc.at[didx], add=True)
        return carry

    lax.fori_loop(0, CPT, body, 0)
    plsc.subcore_barrier()
    pltpu.sync_copy(acc.at[pl.ds(r0, RPT)], out_hbm.at[cid].at[pl.ds(r0, RPT)])


def _make_scatter(F):
    @functools.partial(
        pl.kernel,
        mesh=_mesh,
        out_type=jax.ShapeDtypeStruct((NC, N_PAD, F), jnp.float32),
        scratch_types=[
            pltpu.VMEM((CHUNK,), jnp.int32),
            pltpu.VMEM((CHUNK,), jnp.int32),
            pltpu.VMEM((CHUNK, F), jnp.float32),
            pltpu.VMEM_SHARED((N_PAD, F), jnp.float32),
            pltpu.SemaphoreType.DMA,
        ],
    )
    def _scatter_kernel(h_hbm, src_hbm, dst_hbm, zero_hbm, out_hbm,
                        sidx, didx, rows, acc, sem):
        cid = lax.axis_index("c")
        sid = lax.axis_index("s")
        wid = cid * NS + sid
        r0 = sid * RPT
        pltpu.sync_copy(zero_hbm.at[pl.ds(r0, RPT)], acc.at[pl.ds(r0, RPT)])
        plsc.subcore_barrier()

        def body(j, carry):
            base = (wid * CPT + j) * CHUNK
            pltpu.sync_copy(src_hbm.at[pl.ds(base, CHUNK)], sidx)
            pltpu.sync_copy(dst_hbm.at[pl.ds(base, CHUNK)], didx)
            pltpu.async_copy(h_hbm.at[sidx], rows, sem).wait()
            pltpu.sync_copy(rows, acc.at[didx], add=True)
            return carry

        lax.fori_loop(0, CPT, body, 0)
        plsc.subcore_barrier()
        pltpu.sync_copy(acc.at[pl.ds(r0, RPT)], out_hbm.at[cid].at[pl.ds(r0, RPT)])

    return _scatter_kernel


_scatter_hid = _make_scatter(HID)


def _dinv_from(d0_ref, d1_ref):
    deg = d0_ref[:, 0:1] + d1_ref[:, 0:1] + 1.0
    return 1.0 / jnp.sqrt(deg)


def _first_body(d0_ref, d1_ref, x_ref, w_ref, out_ref):
    dinv = _dinv_from(d0_ref, d1_ref)
    out_ref[...] = (x_ref[...] @ w_ref[...]) * dinv


def _mid_body(d0_ref, d1_ref, sp_ref, hp_ref, b_ref, g_ref, be_ref, w_ref,
              out_ref):
    dinv = _dinv_from(d0_ref, d1_ref)
    y = dinv * (sp_ref[0] + sp_ref[1] + hp_ref[...]) + b_ref[...]
    yr = y[:N]
    m = jnp.mean(yr, axis=0, keepdims=True)
    v = jnp.mean((yr - m) ** 2, axis=0, keepdims=True)
    a = (y - m) / jnp.sqrt(v + EPS) * g_ref[...] + be_ref[...]
    a = jnp.maximum(a, 0.0)
    rows = lax.broadcasted_iota(jnp.int32, a.shape, 0)
    a = jnp.where(rows < N, a, 0.0)
    out_ref[...] = (a @ w_ref[...]) * dinv


def _bn_relu(y, g, b):
    m = jnp.mean(y, axis=0, keepdims=True)
    v = jnp.mean((y - m) ** 2, axis=0, keepdims=True)
    return jnp.maximum((y - m) / jnp.sqrt(v + EPS) * g + b, 0.0)


def _dec_body(dg0_ref, dg1p_ref, sp_ref, hp_ref, b3_ref,
              d1_ref, db1_ref, dg1_ref, dbe1_ref,
              d2_ref, db2_ref, dg2_ref, dbe2_ref,
              d3_ref, db3_ref, xhat_ref, z_ref):
    dinv = _dinv_from(dg0_ref, dg1p_ref)
    zfull = dinv * (sp_ref[0] + sp_ref[1] + hp_ref[...])
    z = zfull[:N, :LAT] + b3_ref[...]
    z_ref[...] = z
    d = _bn_relu(z @ d1_ref[...] + db1_ref[...], dg1_ref[...], dbe1_ref[...])
    d = _bn_relu(d @ d2_ref[...] + db2_ref[...], dg2_ref[...], dbe2_ref[...])
    xhat_ref[...] = d @ d3_ref[...] + db3_ref[...]


def _tc_call(body, out_shape, *args):
    return pl.pallas_call(body, out_shape=out_shape)(*args)


def kernel(x, edge_index, W1, b1, g1, be1, W2, b2, g2, be2, W3, b3,
           D1, db1, dg1, dbe1, D2, db2, dg2, dbe2, D3, db3):
    pad_e = E_PAD - E
    pad_idx = N + (jnp.arange(pad_e, dtype=jnp.int32) % (N_PAD - N))
    src_p = jnp.concatenate([edge_index[0], pad_idx])
    dst_p = jnp.concatenate([edge_index[1], pad_idx])
    x_p = jnp.concatenate([x, jnp.zeros((N_PAD - N, IN), x.dtype)], axis=0)
    z128 = jnp.zeros((N_PAD, HID), jnp.float32)
    ones128 = jnp.ones((CHUNK, HID), jnp.float32)
    r1 = lambda a: a.reshape(1, -1)

    degp = _deg_kernel(dst_p, ones128, z128)
    d0, d1 = degp[0], degp[1]

    h1p = _tc_call(_first_body, jax.ShapeDtypeStruct((N_PAD, HID), jnp.float32),
                   d0, d1, x_p, W1)
    s1 = _scatter_hid(h1p, src_p, dst_p, z128)
    h2p = _tc_call(_mid_body, jax.ShapeDtypeStruct((N_PAD, HID), jnp.float32),
                   d0, d1, s1, h1p, r1(b1), r1(g1), r1(be1), W2)
    s2 = _scatter_hid(h2p, src_p, dst_p, z128)
    W3p = jnp.pad(W3, ((0, 0), (0, HID - LAT)))
    h3p = _tc_call(_mid_body, jax.ShapeDtypeStruct((N_PAD, HID), jnp.float32),
                   d0, d1, s2, h2p, r1(b2), r1(g2), r1(be2), W3p)
    s3 = _scatter_hid(h3p, src_p, dst_p, z128)
    xhat, z = _tc_call(
        _dec_body,
        (jax.ShapeDtypeStruct((N, IN), jnp.float32),
         jax.ShapeDtypeStruct((N, LAT), jnp.float32)),
        d0, d1, s3, h3p, r1(b3),
        D1, r1(db1), r1(dg1), r1(dbe1),
        D2, r1(db2), r1(dg2), r1(dbe2),
        D3, r1(db3))
    return (xhat, z)
